# Initial kernel scaffold; baseline (speedup 1.0000x reference)
import jax
import jax.numpy as jnp
from jax.experimental import pallas as pl


def kernel(x, edge_index, batch, emb, W1, as1, ad1, b1, W2, as2, ad2, b2):
    raise NotImplementedError("write your pallas kernel here")


# speedup vs baseline: 27.2775x; 27.2775x over previous
import dataclasses
import functools

import jax
import jax.numpy as jnp
from jax import lax
from jax.experimental import pallas as pl
from jax.experimental.pallas import tpu as pltpu
from jax.experimental.pallas import tpu_sc as plsc

NC = 2
NS = 16
NW = NC * NS
L = 16
CH = 128
F = 128


_CP = pltpu.CompilerParams()
if "needs_layout_passes" in pltpu.CompilerParams.__dataclass_fields__:
    _CP = dataclasses.replace(_CP, needs_layout_passes=False)


def _leaky(a):
    return jnp.where(a >= 0, a, a * jnp.float32(0.2))


def _sc_edge_pass_vocab(nr, pe, vocab):
    mesh = plsc.VectorSubcoreMesh(core_axis_name="c", subcore_axis_name="s")
    per_tile = pe // NW
    n_chunks = per_tile // CH
    rpt = nr // NS
    vpt = vocab // NS

    @functools.partial(
        pl.kernel,
        out_type=[
            jax.ShapeDtypeStruct((NC, nr, F), jnp.float32),
            jax.ShapeDtypeStruct((NC * nr,), jnp.float32),
        ],
        mesh=mesh,
        scratch_types=[
            pltpu.VMEM((nr,), jnp.int32),
            pltpu.VMEM((vocab,), jnp.float32),
            pltpu.VMEM((vocab,), jnp.float32),
            pltpu.VMEM((CH,), jnp.int32),
            pltpu.VMEM((CH,), jnp.int32),
            pltpu.VMEM((CH,), jnp.int32),
            pltpu.VMEM((CH,), jnp.float32),
            pltpu.VMEM((CH, F), jnp.float32),
            pltpu.VMEM_SHARED((vocab, F), jnp.float32),
            pltpu.VMEM_SHARED((nr, F), jnp.float32),
            pltpu.VMEM_SHARED((nr,), jnp.float32),
            pltpu.SemaphoreType.DMA,
        ],
        compiler_params=_CP,
    )
    def kern(x_hbm, src_hbm, dst_hbm, tbl_hbm, as_hbm, ad_hbm, z2_hbm, z1_hbm,
             out_hbm, den_hbm,
             x_v, as_v, ad_v, src_v, dst_v, xs_v, p_v, stage,
             tbl_sh, out_sh, den_sh, sem):
        c = lax.axis_index("c")
        s = lax.axis_index("s")
        wid = c * NS + s
        pltpu.sync_copy(x_hbm, x_v)
        pltpu.sync_copy(as_hbm, as_v)
        pltpu.sync_copy(ad_hbm, ad_v)
        pltpu.sync_copy(tbl_hbm.at[pl.ds(s * vpt, vpt)],
                        tbl_sh.at[pl.ds(s * vpt, vpt)])
        pltpu.sync_copy(z2_hbm.at[pl.ds(s * rpt, rpt)],
                        out_sh.at[pl.ds(s * rpt, rpt)])
        pltpu.sync_copy(z1_hbm.at[pl.ds(s * rpt, rpt)],
                        den_sh.at[pl.ds(s * rpt, rpt)])
        plsc.subcore_barrier()

        @pl.loop(0, n_chunks)
        def _chunk(ch):
            base = wid * per_tile + ch * CH
            pltpu.sync_copy(src_hbm.at[pl.ds(base, CH)], src_v)
            pltpu.sync_copy(dst_hbm.at[pl.ds(base, CH)], dst_v)

            @pl.loop(0, CH, step=L)
            def _p16(k):
                s16 = src_v[pl.ds(k, L)]
                d16 = dst_v[pl.ds(k, L)]
                xs = plsc.load_gather(x_v, [s16])
                xd = plsc.load_gather(x_v, [d16])
                a = plsc.load_gather(as_v, [xs]) + plsc.load_gather(ad_v, [xd])
                p_v[pl.ds(k, L)] = jnp.exp(_leaky(a))
                xs_v[pl.ds(k, L)] = xs

            pltpu.async_copy(tbl_sh.at[xs_v], stage, sem).wait()

            @pl.loop(0, CH, step=L)
            def _scale(k):
                p16 = p_v[pl.ds(k, L)]
                for lane in range(L):
                    pi = p16[lane]
                    for j in range(F // L):
                        sl = pl.ds(j * L, L)
                        stage[k + lane, sl] = stage[k + lane, sl] * pi

            pltpu.sync_copy(stage, out_sh.at[dst_v], add=True)
            pltpu.sync_copy(p_v, den_sh.at[dst_v], add=True)

        plsc.subcore_barrier()
        pltpu.sync_copy(out_sh.at[pl.ds(s * rpt, rpt)],
                        out_hbm.at[c, pl.ds(s * rpt, rpt)])
        pltpu.sync_copy(den_sh.at[pl.ds(s * rpt, rpt)],
                        den_hbm.at[pl.ds(c * nr + s * rpt, rpt)])

    return kern


def _sc_edge_pass_dense(nr, pe, n):
    mesh = plsc.VectorSubcoreMesh(core_axis_name="c", subcore_axis_name="s")
    per_tile = pe // NW
    n_chunks = per_tile // CH
    rpt = nr // NS

    @functools.partial(
        pl.kernel,
        out_type=[
            jax.ShapeDtypeStruct((NC, nr, F), jnp.float32),
            jax.ShapeDtypeStruct((NC * nr,), jnp.float32),
        ],
        mesh=mesh,
        scratch_types=[
            pltpu.VMEM((nr,), jnp.float32),
            pltpu.VMEM((nr,), jnp.float32),
            pltpu.VMEM((CH,), jnp.int32),
            pltpu.VMEM((CH,), jnp.int32),
            pltpu.VMEM((CH,), jnp.float32),
            pltpu.VMEM((CH, F), jnp.float32),
            pltpu.VMEM_SHARED((nr, F), jnp.float32),
            pltpu.VMEM_SHARED((nr,), jnp.float32),
            pltpu.SemaphoreType.DMA,
        ],
        compiler_params=_CP,
    )
    def kern(src_hbm, dst_hbm, h_hbm, as_hbm, ad_hbm, z2_hbm, z1_hbm,
             out_hbm, den_hbm,
             as_v, ad_v, src_v, dst_v, p_v, rows, out_sh, den_sh, sem):
        c = lax.axis_index("c")
        s = lax.axis_index("s")
        wid = c * NS + s
        pltpu.sync_copy(as_hbm, as_v)
        pltpu.sync_copy(ad_hbm, ad_v)
        pltpu.sync_copy(z2_hbm.at[pl.ds(s * rpt, rpt)],
                        out_sh.at[pl.ds(s * rpt, rpt)])
        pltpu.sync_copy(z1_hbm.at[pl.ds(s * rpt, rpt)],
                        den_sh.at[pl.ds(s * rpt, rpt)])
        plsc.subcore_barrier()

        @pl.loop(0, n_chunks)
        def _chunk(ch):
            base = wid * per_tile + ch * CH
            pltpu.sync_copy(src_hbm.at[pl.ds(base, CH)], src_v)
            pltpu.sync_copy(dst_hbm.at[pl.ds(base, CH)], dst_v)
            gat = pltpu.async_copy(h_hbm.at[src_v], rows, sem)

            @pl.loop(0, CH, step=L)
            def _p16(k):
                s16 = src_v[pl.ds(k, L)]
                d16 = dst_v[pl.ds(k, L)]
                a = plsc.load_gather(as_v, [s16]) + plsc.load_gather(ad_v, [d16])
                p_v[pl.ds(k, L)] = jnp.exp(_leaky(a))

            gat.wait()

            @pl.loop(0, CH, step=L)
            def _scale(k):
                p16 = p_v[pl.ds(k, L)]
                for lane in range(L):
                    pi = p16[lane]
                    for j in range(F // L):
                        sl = pl.ds(j * L, L)
                        rows[k + lane, sl] = rows[k + lane, sl] * pi

            pltpu.sync_copy(rows, out_sh.at[dst_v], add=True)
            pltpu.sync_copy(p_v, den_sh.at[dst_v], add=True)

        plsc.subcore_barrier()
        pltpu.sync_copy(out_sh.at[pl.ds(s * rpt, rpt)],
                        out_hbm.at[c, pl.ds(s * rpt, rpt)])
        pltpu.sync_copy(den_sh.at[pl.ds(s * rpt, rpt)],
                        den_hbm.at[pl.ds(c * nr + s * rpt, rpt)])

    return kern


def _tc_prep(emb, W1, as1, ad1):
    def body(emb_ref, w_ref, a_ref, d_ref, tbl_ref, as_ref, ad_ref):
        tbl = jnp.dot(emb_ref[...], w_ref[...],
                      preferred_element_type=jnp.float32)
        tbl_ref[...] = tbl
        as_ref[...] = jnp.dot(tbl, a_ref[...],
                              preferred_element_type=jnp.float32)
        ad_ref[...] = jnp.dot(tbl, d_ref[...],
                              preferred_element_type=jnp.float32)

    v = emb.shape[0]
    return pl.pallas_call(
        body,
        out_shape=[
            jax.ShapeDtypeStruct((v, F), jnp.float32),
            jax.ShapeDtypeStruct((v, 1), jnp.float32),
            jax.ShapeDtypeStruct((v, 1), jnp.float32),
        ],
    )(emb, W1, as1.reshape(F, 1), ad1.reshape(F, 1))


def _tc_mid(o0, o1, d0, d1, b1, W2, as2, ad2):
    def body(o0_ref, o1_ref, d0_ref, d1_ref, b_ref, w_ref, a_ref, ad_ref,
             h_ref, as_ref, adn_ref):
        den = d0_ref[...] + d1_ref[...] + jnp.float32(1e-16)
        h2 = jnp.maximum(
            (o0_ref[...] + o1_ref[...]) / den + b_ref[...], 0.0)
        hw = jnp.dot(h2, w_ref[...], preferred_element_type=jnp.float32)
        h_ref[...] = hw
        as_ref[...] = jnp.dot(hw, a_ref[...],
                              preferred_element_type=jnp.float32)
        adn_ref[...] = jnp.dot(hw, ad_ref[...],
                               preferred_element_type=jnp.float32)

    n = o0.shape[0]
    return pl.pallas_call(
        body,
        out_shape=[
            jax.ShapeDtypeStruct((n, F), jnp.float32),
            jax.ShapeDtypeStruct((n, 1), jnp.float32),
            jax.ShapeDtypeStruct((n, 1), jnp.float32),
        ],
    )(o0, o1, d0, d1, b1.reshape(1, F), W2, as2.reshape(F, 1),
      ad2.reshape(F, 1))


def _tc_final(o0, o1, d0, d1, b2):
    def body(o0_ref, o1_ref, d0_ref, d1_ref, b_ref, out_ref):
        den = d0_ref[...] + d1_ref[...] + jnp.float32(1e-16)
        out_ref[...] = (o0_ref[...] + o1_ref[...]) / den + b_ref[...]

    n = o0.shape[0]
    return pl.pallas_call(
        body,
        out_shape=jax.ShapeDtypeStruct((n, F), jnp.float32),
    )(o0, o1, d0, d1, b2.reshape(1, F))


def kernel(x, edge_index, batch, emb, W1, as1, ad1, b1, W2, as2, ad2, b2):
    n = x.shape[0]
    e = edge_index.shape[1]
    vocab = emb.shape[0]
    te = e + n
    pe = -(-te // (NW * CH)) * (NW * CH)
    pad = pe - te
    nr = -(-(n + L) // 256) * 256

    loop = jnp.arange(n, dtype=jnp.int32)
    pad_i = jnp.arange(pad, dtype=jnp.int32)
    src = jnp.concatenate([edge_index[0].astype(jnp.int32), loop, pad_i % n])
    dst = jnp.concatenate(
        [edge_index[1].astype(jnp.int32), loop, n + pad_i % (nr - n)])
    x_pad = jnp.concatenate([x.astype(jnp.int32),
                             jnp.zeros((nr - n,), jnp.int32)])
    z2 = jnp.zeros((nr, F), jnp.float32)
    z1 = jnp.zeros((nr,), jnp.float32)

    tbl, as_t, ad_t = _tc_prep(emb, W1, as1, ad1)
    out1, den1 = _sc_edge_pass_vocab(nr, pe, vocab)(
        x_pad, src, dst, tbl, as_t.reshape(vocab), ad_t.reshape(vocab),
        z2, z1)
    hw, asn, adn = _tc_mid(out1[0, :n], out1[1, :n],
                           den1[:n, None], den1[nr:nr + n, None],
                           b1, W2, as2, ad2)
    zpad = jnp.zeros((nr - n,), jnp.float32)
    out2, den2 = _sc_edge_pass_dense(nr, pe, n)(
        src, dst, hw, jnp.concatenate([asn.reshape(n), zpad]),
        jnp.concatenate([adn.reshape(n), zpad]), z2, z1)
    return _tc_final(out2[0, :n], out2[1, :n],
                     den2[:n, None], den2[nr:nr + n, None], b2)


# speedup vs baseline: 44.4837x; 1.6308x over previous
import dataclasses
import functools

import jax
import jax.numpy as jnp
from jax import lax
from jax.experimental import pallas as pl
from jax.experimental.pallas import tpu as pltpu
from jax.experimental.pallas import tpu_sc as plsc

NC = 2
NS = 16
NW = NC * NS
L = 16
CH = 96
NB = 3
F = 128


_CP = pltpu.CompilerParams()
if "needs_layout_passes" in pltpu.CompilerParams.__dataclass_fields__:
    _CP = dataclasses.replace(_CP, needs_layout_passes=False)


def _leaky(a):
    return jnp.where(a >= 0, a, a * jnp.float32(0.2))


def _sc_edge_pass_real(nr, pe):
    mesh = plsc.VectorSubcoreMesh(core_axis_name="c", subcore_axis_name="s")
    per_tile = pe // NW
    nchunks = per_tile // CH
    rpt = nr // NS

    scratch = (
        [pltpu.VMEM((2 * CH,), jnp.int32) for _ in range(NB)]
        + [pltpu.VMEM((CH, F), jnp.float32) for _ in range(NB)]
        + [pltpu.VMEM((CH,), jnp.float32) for _ in range(NB)]
        + [pltpu.VMEM((CH,), jnp.float32) for _ in range(NB)]
        + [pltpu.VMEM((CH,), jnp.float32) for _ in range(NB)]
        + [pltpu.VMEM((CH,), jnp.int32) for _ in range(NB)]
        + [pltpu.VMEM_SHARED((nr, F), jnp.float32),
           pltpu.VMEM_SHARED((nr,), jnp.float32)]
        + [pltpu.SemaphoreType.DMA for _ in range(3 * NB)]
    )

    @functools.partial(
        pl.kernel,
        out_type=[
            jax.ShapeDtypeStruct((NC, nr, F), jnp.float32),
            jax.ShapeDtypeStruct((NC * nr,), jnp.float32),
        ],
        mesh=mesh,
        scratch_types=scratch,
        compiler_params=_CP,
    )
    def kern(sd_hbm, h_hbm, as_hbm, ad_hbm, z2_hbm, z1_hbm,
             out_hbm, den_hbm, *bufs):
        idx = bufs[0:NB]
        rows = bufs[NB:2 * NB]
        ase = bufs[2 * NB:3 * NB]
        ade = bufs[3 * NB:4 * NB]
        pv = bufs[4 * NB:5 * NB]
        dsc = bufs[5 * NB:6 * NB]
        out_sh = bufs[6 * NB]
        den_sh = bufs[6 * NB + 1]
        sem_i = bufs[6 * NB + 2:6 * NB + 2 + NB]
        sem_g = bufs[6 * NB + 2 + NB:6 * NB + 2 + 2 * NB]
        sem_o = bufs[6 * NB + 2 + 2 * NB:6 * NB + 2 + 3 * NB]

        c = lax.axis_index("c")
        s = lax.axis_index("s")
        wid = c * NS + s
        base0 = wid * nchunks

        def idx_copy(ch_i, b):
            return pltpu.make_async_copy(
                sd_hbm.at[pl.ds((base0 + ch_i) * 2 * CH, 2 * CH)],
                idx[b], sem_i[b])

        def gather_copies(b):
            srcs = idx[b].at[pl.ds(0, CH)]
            dsts = idx[b].at[pl.ds(CH, CH)]
            return (
                pltpu.make_async_copy(h_hbm.at[srcs], rows[b], sem_g[b]),
                pltpu.make_async_copy(as_hbm.at[srcs], ase[b], sem_g[b]),
                pltpu.make_async_copy(ad_hbm.at[dsts], ade[b], sem_g[b]),
            )

        def scatter_copies(b):
            return (
                pltpu.make_async_copy(rows[b], out_sh.at[dsc[b]], sem_o[b]),
                pltpu.make_async_copy(pv[b], den_sh.at[dsc[b]], sem_o[b]),
            )

        def issue(copies, add=False):
            for cp in copies:
                cp.start(add=add)

        def drain(copies):
            for cp in copies:
                cp.wait()

        def process(ch_i, b):
            bn = (b + 1) % NB
            bz = (b + 2) % NB
            drain([idx_copy(ch_i + 1, bn)])
            pl.when(ch_i >= 2)(lambda: drain(scatter_copies(bn)))
            issue(gather_copies(bn))
            pl.when(ch_i + 2 < nchunks)(lambda: issue([idx_copy(ch_i + 2, bz)]))
            drain(gather_copies(b))
            @pl.loop(0, CH, step=L)
            def _p16(k):
                a = ase[b][pl.ds(k, L)] + ade[b][pl.ds(k, L)]
                pv[b][pl.ds(k, L)] = jnp.exp(_leaky(a))
                dsc[b][pl.ds(k, L)] = idx[b][pl.ds(CH + k, L)]

            @pl.loop(0, CH, step=L)
            def _scale(k):
                p16 = pv[b][pl.ds(k, L)]
                for lane in range(L):
                    pi = p16[lane]
                    for j in range(F // L):
                        sl = pl.ds(j * L, L)
                        rows[b][k + lane, sl] = rows[b][k + lane, sl] * pi

            issue(scatter_copies(b), add=True)

        pltpu.sync_copy(z2_hbm.at[pl.ds(s * rpt, rpt)],
                        out_sh.at[pl.ds(s * rpt, rpt)])
        pltpu.sync_copy(z1_hbm.at[pl.ds(s * rpt, rpt)],
                        den_sh.at[pl.ds(s * rpt, rpt)])
        plsc.subcore_barrier()

        issue([idx_copy(0, 0)])
        issue([idx_copy(1, 1)])
        drain([idx_copy(0, 0)])
        issue(gather_copies(0))

        @pl.loop(0, nchunks - 1, step=NB)
        def _blk(ci):
            for g in range(NB):
                pl.when(ci + g < nchunks - 1)(
                    functools.partial(process, ci + g, g))

        last = nchunks - 1
        bl = (last) % NB
        drain(gather_copies(bl))

        @pl.loop(0, CH, step=L)
        def _p16t(k):
            a = ase[bl][pl.ds(k, L)] + ade[bl][pl.ds(k, L)]
            pv[bl][pl.ds(k, L)] = jnp.exp(_leaky(a))
            dsc[bl][pl.ds(k, L)] = idx[bl][pl.ds(CH + k, L)]

        @pl.loop(0, CH, step=L)
        def _scalet(k):
            p16 = pv[bl][pl.ds(k, L)]
            for lane in range(L):
                pi = p16[lane]
                for j in range(F // L):
                    sl = pl.ds(j * L, L)
                    rows[bl][k + lane, sl] = rows[bl][k + lane, sl] * pi

        issue(scatter_copies(bl), add=True)
        drain(scatter_copies((last - 2) % NB))
        drain(scatter_copies((last - 1) % NB))
        drain(scatter_copies(bl))

        plsc.subcore_barrier()
        pltpu.sync_copy(out_sh.at[pl.ds(s * rpt, rpt)],
                        out_hbm.at[c, pl.ds(s * rpt, rpt)])
        pltpu.sync_copy(den_sh.at[pl.ds(s * rpt, rpt)],
                        den_hbm.at[pl.ds(c * nr + s * rpt, rpt)])

    return kern


def _tc_prep(x2d, emb, W1, as1, ad1):
    n = x2d.shape[0]
    v, h = emb.shape
    blk = 2000

    def body(x_ref, emb_ref, w_ref, a_ref, d_ref, h_ref, as_ref, ad_ref):
        tbl = jnp.dot(emb_ref[...], w_ref[...],
                      preferred_element_type=jnp.float32)
        oh = (x_ref[...] == lax.broadcasted_iota(jnp.int32, (blk, v), 1)
              ).astype(jnp.float32)
        hw = jnp.dot(oh, tbl, precision=lax.Precision.HIGHEST,
                     preferred_element_type=jnp.float32)
        h_ref[...] = hw
        as_ref[...] = jnp.dot(hw, a_ref[...],
                              preferred_element_type=jnp.float32)
        ad_ref[...] = jnp.dot(hw, d_ref[...],
                              preferred_element_type=jnp.float32)

    return pl.pallas_call(
        body,
        grid=(n // blk,),
        in_specs=[
            pl.BlockSpec((blk, 1), lambda i: (i, 0)),
            pl.BlockSpec((v, h), lambda i: (0, 0)),
            pl.BlockSpec((h, h), lambda i: (0, 0)),
            pl.BlockSpec((h, 1), lambda i: (0, 0)),
            pl.BlockSpec((h, 1), lambda i: (0, 0)),
        ],
        out_specs=[
            pl.BlockSpec((blk, F), lambda i: (i, 0)),
            pl.BlockSpec((blk, 1), lambda i: (i, 0)),
            pl.BlockSpec((blk, 1), lambda i: (i, 0)),
        ],
        out_shape=[
            jax.ShapeDtypeStruct((n, F), jnp.float32),
            jax.ShapeDtypeStruct((n, 1), jnp.float32),
            jax.ShapeDtypeStruct((n, 1), jnp.float32),
        ],
    )(x2d, emb, W1, as1.reshape(h, 1), ad1.reshape(h, 1))


def _tc_mid(o0, o1, d0, d1, b1, W2, as2, ad2):
    def body(o0_ref, o1_ref, d0_ref, d1_ref, b_ref, w_ref, a_ref, ad_ref,
             h_ref, as_ref, adn_ref):
        den = d0_ref[...] + d1_ref[...] + jnp.float32(1e-16)
        h2 = jnp.maximum(
            (o0_ref[...] + o1_ref[...]) / den + b_ref[...], 0.0)
        hw = jnp.dot(h2, w_ref[...], preferred_element_type=jnp.float32)
        h_ref[...] = hw
        as_ref[...] = jnp.dot(hw, a_ref[...],
                              preferred_element_type=jnp.float32)
        adn_ref[...] = jnp.dot(hw, ad_ref[...],
                               preferred_element_type=jnp.float32)

    n = o0.shape[0]
    return pl.pallas_call(
        body,
        out_shape=[
            jax.ShapeDtypeStruct((n, F), jnp.float32),
            jax.ShapeDtypeStruct((n, 1), jnp.float32),
            jax.ShapeDtypeStruct((n, 1), jnp.float32),
        ],
    )(o0, o1, d0, d1, b1.reshape(1, F), W2, as2.reshape(F, 1),
      ad2.reshape(F, 1))


def _tc_final(o0, o1, d0, d1, b2):
    def body(o0_ref, o1_ref, d0_ref, d1_ref, b_ref, out_ref):
        den = d0_ref[...] + d1_ref[...] + jnp.float32(1e-16)
        out_ref[...] = (o0_ref[...] + o1_ref[...]) / den + b_ref[...]

    n = o0.shape[0]
    return pl.pallas_call(
        body,
        out_shape=jax.ShapeDtypeStruct((n, F), jnp.float32),
    )(o0, o1, d0, d1, b2.reshape(1, F))


def kernel(x, edge_index, batch, emb, W1, as1, ad1, b1, W2, as2, ad2, b2):
    n = x.shape[0]
    e = edge_index.shape[1]
    te = e + n
    unit = NW * CH * NB
    pe = -(-te // unit) * unit
    pad = pe - te
    nr = -(-(n + L) // 256) * 256

    loop = jnp.arange(n, dtype=jnp.int32)
    pad_i = jnp.arange(pad, dtype=jnp.int32)
    src = jnp.concatenate([edge_index[0].astype(jnp.int32), loop, pad_i % n])
    dst = jnp.concatenate(
        [edge_index[1].astype(jnp.int32), loop, n + pad_i % (nr - n)])
    sd = jnp.concatenate(
        [src.reshape(pe // CH, CH), dst.reshape(pe // CH, CH)],
        axis=1).reshape(-1)
    z2 = jnp.zeros((nr, F), jnp.float32)
    z1 = jnp.zeros((nr,), jnp.float32)
    zpad = jnp.zeros((nr - n,), jnp.float32)

    h1, as1n, ad1n = _tc_prep(x.reshape(n, 1).astype(jnp.int32),
                              emb, W1, as1, ad1)
    edge_pass = _sc_edge_pass_real(nr, pe)
    out1, den1 = edge_pass(
        sd, h1, jnp.concatenate([as1n.reshape(n), zpad]),
        jnp.concatenate([ad1n.reshape(n), zpad]), z2, z1)
    hw, asn, adn = _tc_mid(out1[0, :n], out1[1, :n],
                           den1[:n, None], den1[nr:nr + n, None],
                           b1, W2, as2, ad2)
    out2, den2 = edge_pass(
        sd, hw, jnp.concatenate([asn.reshape(n), zpad]),
        jnp.concatenate([adn.reshape(n), zpad]), z2, z1)
    return _tc_final(out2[0, :n], out2[1, :n],
                     den2[:n, None], den2[nr:nr + n, None], b2)


# speedup vs baseline: 47.8940x; 1.0767x over previous
import dataclasses
import functools

import jax
import jax.numpy as jnp
from jax import lax
from jax.experimental import pallas as pl
from jax.experimental.pallas import tpu as pltpu
from jax.experimental.pallas import tpu_sc as plsc

NC = 2
NS = 16
NW = NC * NS
L = 16
CH = 112
NB = 3
F = 128


_CP = pltpu.CompilerParams()
if "needs_layout_passes" in pltpu.CompilerParams.__dataclass_fields__:
    _CP = dataclasses.replace(_CP, needs_layout_passes=False)


def _leaky(a):
    return jnp.where(a >= 0, a, a * jnp.float32(0.2))


def _sc_edge_pass_real(nr, pe):
    mesh = plsc.VectorSubcoreMesh(core_axis_name="c", subcore_axis_name="s")
    per_tile = pe // NW
    nchunks = per_tile // CH
    rpt = nr // NS

    scratch = (
        [pltpu.VMEM((2 * CH,), jnp.int32) for _ in range(NB)]
        + [pltpu.VMEM((CH, F), jnp.float32) for _ in range(NB)]
        + [pltpu.VMEM((CH,), jnp.float32) for _ in range(NB)]
        + [pltpu.VMEM((CH,), jnp.float32) for _ in range(NB)]
        + [pltpu.VMEM((CH,), jnp.float32) for _ in range(NB)]
        + [pltpu.VMEM((CH,), jnp.int32) for _ in range(NB)]
        + [pltpu.VMEM_SHARED((nr, F), jnp.float32),
           pltpu.VMEM_SHARED((nr,), jnp.float32)]
        + [pltpu.SemaphoreType.DMA for _ in range(4 * NB)]
    )

    @functools.partial(
        pl.kernel,
        out_type=[
            jax.ShapeDtypeStruct((NC, nr, F), jnp.float32),
            jax.ShapeDtypeStruct((NC * nr,), jnp.float32),
        ],
        mesh=mesh,
        scratch_types=scratch,
        compiler_params=_CP,
    )
    def kern(sd_hbm, h_hbm, as_hbm, ad_hbm, z2_hbm, z1_hbm,
             out_hbm, den_hbm, *bufs):
        idx = bufs[0:NB]
        rows = bufs[NB:2 * NB]
        ase = bufs[2 * NB:3 * NB]
        ade = bufs[3 * NB:4 * NB]
        pv = bufs[4 * NB:5 * NB]
        dsc = bufs[5 * NB:6 * NB]
        out_sh = bufs[6 * NB]
        den_sh = bufs[6 * NB + 1]
        sem_i = bufs[6 * NB + 2:6 * NB + 2 + NB]
        sem_g = bufs[6 * NB + 2 + NB:6 * NB + 2 + 2 * NB]
        sem_o = bufs[6 * NB + 2 + 2 * NB:6 * NB + 2 + 3 * NB]
        sem_a = bufs[6 * NB + 2 + 3 * NB:6 * NB + 2 + 4 * NB]

        c = lax.axis_index("c")
        s = lax.axis_index("s")
        wid = c * NS + s
        base0 = wid * nchunks

        def idx_copy(ch_i, b):
            return pltpu.make_async_copy(
                sd_hbm.at[pl.ds((base0 + ch_i) * 2 * CH, 2 * CH)],
                idx[b], sem_i[b])

        def row_copy(b):
            return (
                pltpu.make_async_copy(h_hbm.at[idx[b].at[pl.ds(0, CH)]],
                                      rows[b], sem_g[b]),
            )

        def small_copies(b):
            return (
                pltpu.make_async_copy(as_hbm.at[idx[b].at[pl.ds(0, CH)]],
                                      ase[b], sem_a[b]),
                pltpu.make_async_copy(ad_hbm.at[idx[b].at[pl.ds(CH, CH)]],
                                      ade[b], sem_a[b]),
            )

        def gather_copies(b):
            return small_copies(b) + row_copy(b)

        def row_scat(b):
            return (
                pltpu.make_async_copy(rows[b], out_sh.at[dsc[b]], sem_o[b]),
            )

        def den_scat(b):
            return (
                pltpu.make_async_copy(pv[b], den_sh.at[dsc[b]], sem_o[b]),
            )

        def scatter_copies(b):
            return row_scat(b) + den_scat(b)

        def issue(copies, add=False):
            for cp in copies:
                cp.start(add=add)

        def drain(copies):
            for cp in copies:
                cp.wait()

        def process(ch_i, b):
            bn = (b + 1) % NB
            bz = (b + 2) % NB
            drain([idx_copy(ch_i + 1, bn)])
            pl.when(ch_i >= 2)(lambda: drain(scatter_copies(bn)))
            issue(gather_copies(bn))
            pl.when(ch_i + 2 < nchunks)(lambda: issue([idx_copy(ch_i + 2, bz)]))
            drain(small_copies(b))
            @pl.loop(0, CH, step=L)
            def _p16(k):
                a = ase[b][pl.ds(k, L)] + ade[b][pl.ds(k, L)]
                pv[b][pl.ds(k, L)] = jnp.exp(_leaky(a))
                dsc[b][pl.ds(k, L)] = idx[b][pl.ds(CH + k, L)]

            issue(den_scat(b), add=True)
            drain(row_copy(b))
            @pl.loop(0, CH, step=L)
            def _scale(k):
                p16 = pv[b][pl.ds(k, L)]
                for lane in range(L):
                    pi = p16[lane]
                    for j in range(F // L):
                        sl = pl.ds(j * L, L)
                        rows[b][k + lane, sl] = rows[b][k + lane, sl] * pi

            issue(row_scat(b), add=True)

        pltpu.sync_copy(z2_hbm.at[pl.ds(s * rpt, rpt)],
                        out_sh.at[pl.ds(s * rpt, rpt)])
        pltpu.sync_copy(z1_hbm.at[pl.ds(s * rpt, rpt)],
                        den_sh.at[pl.ds(s * rpt, rpt)])
        plsc.subcore_barrier()

        issue([idx_copy(0, 0)])
        issue([idx_copy(1, 1)])
        drain([idx_copy(0, 0)])
        issue(gather_copies(0))

        @pl.loop(0, nchunks - 1, step=NB)
        def _blk(ci):
            for g in range(NB):
                pl.when(ci + g < nchunks - 1)(
                    functools.partial(process, ci + g, g))

        last = nchunks - 1
        bl = (last) % NB
        drain(small_copies(bl))

        @pl.loop(0, CH, step=L)
        def _p16t(k):
            a = ase[bl][pl.ds(k, L)] + ade[bl][pl.ds(k, L)]
            pv[bl][pl.ds(k, L)] = jnp.exp(_leaky(a))
            dsc[bl][pl.ds(k, L)] = idx[bl][pl.ds(CH + k, L)]

        issue(den_scat(bl), add=True)
        drain(row_copy(bl))

        @pl.loop(0, CH, step=L)
        def _scalet(k):
            p16 = pv[bl][pl.ds(k, L)]
            for lane in range(L):
                pi = p16[lane]
                for j in range(F // L):
                    sl = pl.ds(j * L, L)
                    rows[bl][k + lane, sl] = rows[bl][k + lane, sl] * pi

        issue(row_scat(bl), add=True)
        drain(scatter_copies((last - 2) % NB))
        drain(scatter_copies((last - 1) % NB))
        drain(scatter_copies(bl))

        plsc.subcore_barrier()
        pltpu.sync_copy(out_sh.at[pl.ds(s * rpt, rpt)],
                        out_hbm.at[c, pl.ds(s * rpt, rpt)])
        pltpu.sync_copy(den_sh.at[pl.ds(s * rpt, rpt)],
                        den_hbm.at[pl.ds(c * nr + s * rpt, rpt)])

    return kern


def _tc_prep(x2d, emb, W1, as1, ad1):
    n = x2d.shape[0]
    v, h = emb.shape
    blk = 2000

    def body(x_ref, emb_ref, w_ref, a_ref, d_ref, h_ref, as_ref, ad_ref):
        tbl = jnp.dot(emb_ref[...], w_ref[...],
                      preferred_element_type=jnp.float32)
        oh = (x_ref[...] == lax.broadcasted_iota(jnp.int32, (blk, v), 1)
              ).astype(jnp.float32)
        hw = jnp.dot(oh, tbl, precision=lax.Precision.HIGHEST,
                     preferred_element_type=jnp.float32)
        h_ref[...] = hw
        as_ref[...] = jnp.dot(hw, a_ref[...],
                              preferred_element_type=jnp.float32)
        ad_ref[...] = jnp.dot(hw, d_ref[...],
                              preferred_element_type=jnp.float32)

    return pl.pallas_call(
        body,
        grid=(n // blk,),
        in_specs=[
            pl.BlockSpec((blk, 1), lambda i: (i, 0)),
            pl.BlockSpec((v, h), lambda i: (0, 0)),
            pl.BlockSpec((h, h), lambda i: (0, 0)),
            pl.BlockSpec((h, 1), lambda i: (0, 0)),
            pl.BlockSpec((h, 1), lambda i: (0, 0)),
        ],
        out_specs=[
            pl.BlockSpec((blk, F), lambda i: (i, 0)),
            pl.BlockSpec((blk, 1), lambda i: (i, 0)),
            pl.BlockSpec((blk, 1), lambda i: (i, 0)),
        ],
        out_shape=[
            jax.ShapeDtypeStruct((n, F), jnp.float32),
            jax.ShapeDtypeStruct((n, 1), jnp.float32),
            jax.ShapeDtypeStruct((n, 1), jnp.float32),
        ],
    )(x2d, emb, W1, as1.reshape(h, 1), ad1.reshape(h, 1))


def _tc_mid(o0, o1, d0, d1, b1, W2, as2, ad2):
    def body(o0_ref, o1_ref, d0_ref, d1_ref, b_ref, w_ref, a_ref, ad_ref,
             h_ref, as_ref, adn_ref):
        den = d0_ref[...] + d1_ref[...] + jnp.float32(1e-16)
        h2 = jnp.maximum(
            (o0_ref[...] + o1_ref[...]) / den + b_ref[...], 0.0)
        hw = jnp.dot(h2, w_ref[...], preferred_element_type=jnp.float32)
        h_ref[...] = hw
        as_ref[...] = jnp.dot(hw, a_ref[...],
                              preferred_element_type=jnp.float32)
        adn_ref[...] = jnp.dot(hw, ad_ref[...],
                               preferred_element_type=jnp.float32)

    n = o0.shape[0]
    return pl.pallas_call(
        body,
        out_shape=[
            jax.ShapeDtypeStruct((n, F), jnp.float32),
            jax.ShapeDtypeStruct((n, 1), jnp.float32),
            jax.ShapeDtypeStruct((n, 1), jnp.float32),
        ],
    )(o0, o1, d0, d1, b1.reshape(1, F), W2, as2.reshape(F, 1),
      ad2.reshape(F, 1))


def _tc_final(o0, o1, d0, d1, b2):
    def body(o0_ref, o1_ref, d0_ref, d1_ref, b_ref, out_ref):
        den = d0_ref[...] + d1_ref[...] + jnp.float32(1e-16)
        out_ref[...] = (o0_ref[...] + o1_ref[...]) / den + b_ref[...]

    n = o0.shape[0]
    return pl.pallas_call(
        body,
        out_shape=jax.ShapeDtypeStruct((n, F), jnp.float32),
    )(o0, o1, d0, d1, b2.reshape(1, F))


def kernel(x, edge_index, batch, emb, W1, as1, ad1, b1, W2, as2, ad2, b2):
    n = x.shape[0]
    e = edge_index.shape[1]
    te = e + n
    unit = NW * CH * NB
    pe = -(-te // unit) * unit
    pad = pe - te
    nr = -(-(n + L) // 256) * 256

    loop = jnp.arange(n, dtype=jnp.int32)
    pad_i = jnp.arange(pad, dtype=jnp.int32)
    src = jnp.concatenate([edge_index[0].astype(jnp.int32), loop, pad_i % n])
    dst = jnp.concatenate(
        [edge_index[1].astype(jnp.int32), loop, n + pad_i % (nr - n)])
    sd = jnp.concatenate(
        [src.reshape(pe // CH, CH), dst.reshape(pe // CH, CH)],
        axis=1).reshape(-1)
    z2 = jnp.zeros((nr, F), jnp.float32)
    z1 = jnp.zeros((nr,), jnp.float32)
    zpad = jnp.zeros((nr - n,), jnp.float32)

    h1, as1n, ad1n = _tc_prep(x.reshape(n, 1).astype(jnp.int32),
                              emb, W1, as1, ad1)
    edge_pass = _sc_edge_pass_real(nr, pe)
    out1, den1 = edge_pass(
        sd, h1, jnp.concatenate([as1n.reshape(n), zpad]),
        jnp.concatenate([ad1n.reshape(n), zpad]), z2, z1)
    hw, asn, adn = _tc_mid(out1[0, :n], out1[1, :n],
                           den1[:n, None], den1[nr:nr + n, None],
                           b1, W2, as2, ad2)
    out2, den2 = edge_pass(
        sd, hw, jnp.concatenate([asn.reshape(n), zpad]),
        jnp.concatenate([adn.reshape(n), zpad]), z2, z1)
    return _tc_final(out2[0, :n], out2[1, :n],
                     den2[:n, None], den2[nr:nr + n, None], b2)


# speedup vs baseline: 47.9261x; 1.0007x over previous
import dataclasses
import functools

import jax
import jax.numpy as jnp
from jax import lax
from jax.experimental import pallas as pl
from jax.experimental.pallas import tpu as pltpu
from jax.experimental.pallas import tpu_sc as plsc

NC = 2
NS = 16
NW = NC * NS
L = 16
CH = 112
NB = 3
F = 128


_CP = pltpu.CompilerParams()
if "needs_layout_passes" in pltpu.CompilerParams.__dataclass_fields__:
    _CP = dataclasses.replace(_CP, needs_layout_passes=False)


def _leaky(a):
    return jnp.where(a >= 0, a, a * jnp.float32(0.2))


def _sc_edge_pass_real(nr, pe):
    mesh = plsc.VectorSubcoreMesh(core_axis_name="c", subcore_axis_name="s")
    per_tile = pe // NW
    nchunks = per_tile // CH
    rpt = nr // NS

    scratch = (
        [pltpu.VMEM((2 * CH,), jnp.int32) for _ in range(NB)]
        + [pltpu.VMEM((CH, F), jnp.float32) for _ in range(NB)]
        + [pltpu.VMEM((CH,), jnp.float32) for _ in range(NB)]
        + [pltpu.VMEM((CH,), jnp.float32) for _ in range(NB)]
        + [pltpu.VMEM((CH,), jnp.float32) for _ in range(NB)]
        + [pltpu.VMEM((CH,), jnp.int32) for _ in range(NB)]
        + [pltpu.VMEM_SHARED((nr, F), jnp.float32),
           pltpu.VMEM_SHARED((nr,), jnp.float32)]
        + [pltpu.SemaphoreType.DMA for _ in range(4 * NB)]
    )

    @functools.partial(
        pl.kernel,
        out_type=[
            jax.ShapeDtypeStruct((NC, nr, F), jnp.float32),
            jax.ShapeDtypeStruct((NC * nr,), jnp.float32),
        ],
        mesh=mesh,
        scratch_types=scratch,
        compiler_params=_CP,
    )
    def kern(sd_hbm, h_hbm, as_hbm, ad_hbm, z2_hbm, z1_hbm,
             out_hbm, den_hbm, *bufs):
        idx = bufs[0:NB]
        rows = bufs[NB:2 * NB]
        ase = bufs[2 * NB:3 * NB]
        ade = bufs[3 * NB:4 * NB]
        pv = bufs[4 * NB:5 * NB]
        dsc = bufs[5 * NB:6 * NB]
        out_sh = bufs[6 * NB]
        den_sh = bufs[6 * NB + 1]
        sem_i = bufs[6 * NB + 2:6 * NB + 2 + NB]
        sem_g = bufs[6 * NB + 2 + NB:6 * NB + 2 + 2 * NB]
        sem_o = bufs[6 * NB + 2 + 2 * NB:6 * NB + 2 + 3 * NB]
        sem_a = bufs[6 * NB + 2 + 3 * NB:6 * NB + 2 + 4 * NB]

        c = lax.axis_index("c")
        s = lax.axis_index("s")
        wid = c * NS + s
        base0 = wid * nchunks

        def idx_copy(ch_i, b):
            return pltpu.make_async_copy(
                sd_hbm.at[pl.ds((base0 + ch_i) * 2 * CH, 2 * CH)],
                idx[b], sem_i[b])

        def row_copy(b):
            return (
                pltpu.make_async_copy(h_hbm.at[idx[b].at[pl.ds(0, CH)]],
                                      rows[b], sem_g[b]),
            )

        def small_copies(b):
            return (
                pltpu.make_async_copy(as_hbm.at[idx[b].at[pl.ds(0, CH)]],
                                      ase[b], sem_a[b]),
                pltpu.make_async_copy(ad_hbm.at[idx[b].at[pl.ds(CH, CH)]],
                                      ade[b], sem_a[b]),
            )

        def gather_copies(b):
            return small_copies(b) + row_copy(b)

        def row_scat(b):
            return (
                pltpu.make_async_copy(rows[b], out_sh.at[dsc[b]], sem_o[b]),
            )

        def den_scat(b):
            return (
                pltpu.make_async_copy(pv[b], den_sh.at[dsc[b]], sem_o[b]),
            )

        def scatter_copies(b):
            return row_scat(b) + den_scat(b)

        def issue(copies, add=False):
            for cp in copies:
                cp.start(add=add)

        def drain(copies):
            for cp in copies:
                cp.wait()

        def process(ch_i, b):
            bn = (b + 1) % NB
            bz = (b + 2) % NB
            drain([idx_copy(ch_i + 1, bn)])
            pl.when(ch_i >= 2)(lambda: drain(scatter_copies(bn)))
            issue(gather_copies(bn))
            pl.when(ch_i + 2 < nchunks)(lambda: issue([idx_copy(ch_i + 2, bz)]))
            drain(small_copies(b))
            @pl.loop(0, CH, step=L)
            def _p16(k):
                a = ase[b][pl.ds(k, L)] + ade[b][pl.ds(k, L)]
                pv[b][pl.ds(k, L)] = jnp.exp(_leaky(a))
                dsc[b][pl.ds(k, L)] = idx[b][pl.ds(CH + k, L)]

            issue(den_scat(b), add=True)
            drain(row_copy(b))
            @pl.loop(0, CH, step=L)
            def _scale(k):
                p16 = pv[b][pl.ds(k, L)]
                for lane in range(L):
                    pi = p16[lane]
                    for j in range(F // L):
                        sl = pl.ds(j * L, L)
                        rows[b][k + lane, sl] = rows[b][k + lane, sl] * pi

            issue(row_scat(b), add=True)

        pltpu.sync_copy(z2_hbm, out_sh.at[pl.ds(s * rpt, rpt)])
        pltpu.sync_copy(z1_hbm, den_sh.at[pl.ds(s * rpt, rpt)])
        plsc.subcore_barrier()

        issue([idx_copy(0, 0)])
        issue([idx_copy(1, 1)])
        drain([idx_copy(0, 0)])
        issue(gather_copies(0))

        @pl.loop(0, nchunks - 1, step=NB)
        def _blk(ci):
            for g in range(NB):
                pl.when(ci + g < nchunks - 1)(
                    functools.partial(process, ci + g, g))

        last = nchunks - 1
        bl = (last) % NB
        drain(small_copies(bl))

        @pl.loop(0, CH, step=L)
        def _p16t(k):
            a = ase[bl][pl.ds(k, L)] + ade[bl][pl.ds(k, L)]
            pv[bl][pl.ds(k, L)] = jnp.exp(_leaky(a))
            dsc[bl][pl.ds(k, L)] = idx[bl][pl.ds(CH + k, L)]

        issue(den_scat(bl), add=True)
        drain(row_copy(bl))

        @pl.loop(0, CH, step=L)
        def _scalet(k):
            p16 = pv[bl][pl.ds(k, L)]
            for lane in range(L):
                pi = p16[lane]
                for j in range(F // L):
                    sl = pl.ds(j * L, L)
                    rows[bl][k + lane, sl] = rows[bl][k + lane, sl] * pi

        issue(row_scat(bl), add=True)
        drain(scatter_copies((last - 2) % NB))
        drain(scatter_copies((last - 1) % NB))
        drain(scatter_copies(bl))

        plsc.subcore_barrier()
        pltpu.sync_copy(out_sh.at[pl.ds(s * rpt, rpt)],
                        out_hbm.at[c, pl.ds(s * rpt, rpt)])
        pltpu.sync_copy(den_sh.at[pl.ds(s * rpt, rpt)],
                        den_hbm.at[pl.ds(c * nr + s * rpt, rpt)])

    return kern


def _tc_prep(x2d, emb, W1, as1, ad1):
    n = x2d.shape[0]
    v, h = emb.shape
    blk = 2000

    def body(x_ref, emb_ref, w_ref, a_ref, d_ref, h_ref, as_ref, ad_ref):
        tbl = jnp.dot(emb_ref[...], w_ref[...],
                      preferred_element_type=jnp.float32)
        oh = (x_ref[...] == lax.broadcasted_iota(jnp.int32, (blk, v), 1)
              ).astype(jnp.float32)
        hw = jnp.dot(oh, tbl, precision=lax.Precision.HIGHEST,
                     preferred_element_type=jnp.float32)
        h_ref[...] = hw
        as_ref[...] = jnp.dot(hw, a_ref[...],
                              preferred_element_type=jnp.float32)
        ad_ref[...] = jnp.dot(hw, d_ref[...],
                              preferred_element_type=jnp.float32)

    return pl.pallas_call(
        body,
        grid=(n // blk,),
        in_specs=[
            pl.BlockSpec((blk, 1), lambda i: (i, 0)),
            pl.BlockSpec((v, h), lambda i: (0, 0)),
            pl.BlockSpec((h, h), lambda i: (0, 0)),
            pl.BlockSpec((h, 1), lambda i: (0, 0)),
            pl.BlockSpec((h, 1), lambda i: (0, 0)),
        ],
        out_specs=[
            pl.BlockSpec((blk, F), lambda i: (i, 0)),
            pl.BlockSpec((blk, 1), lambda i: (i, 0)),
            pl.BlockSpec((blk, 1), lambda i: (i, 0)),
        ],
        out_shape=[
            jax.ShapeDtypeStruct((n, F), jnp.float32),
            jax.ShapeDtypeStruct((n, 1), jnp.float32),
            jax.ShapeDtypeStruct((n, 1), jnp.float32),
        ],
    )(x2d, emb, W1, as1.reshape(h, 1), ad1.reshape(h, 1))


def _tc_mid(o_full, d0, d1, b1, W2, as2, ad2, n):
    blk = 2000

    def body(o0_ref, o1_ref, d0_ref, d1_ref, b_ref, w_ref, a_ref, ad_ref,
             h_ref, as_ref, adn_ref):
        den = d0_ref[...] + d1_ref[...] + jnp.float32(1e-16)
        h2 = jnp.maximum(
            (o0_ref[0] + o1_ref[0]) / den + b_ref[...], 0.0)
        hw = jnp.dot(h2, w_ref[...], preferred_element_type=jnp.float32)
        h_ref[...] = hw
        as_ref[...] = jnp.dot(hw, a_ref[...],
                              preferred_element_type=jnp.float32)
        adn_ref[...] = jnp.dot(hw, ad_ref[...],
                               preferred_element_type=jnp.float32)

    return pl.pallas_call(
        body,
        grid=(n // blk,),
        in_specs=[
            pl.BlockSpec((1, blk, F), lambda i: (0, i, 0)),
            pl.BlockSpec((1, blk, F), lambda i: (1, i, 0)),
            pl.BlockSpec((blk, 1), lambda i: (i, 0)),
            pl.BlockSpec((blk, 1), lambda i: (i, 0)),
            pl.BlockSpec((1, F), lambda i: (0, 0)),
            pl.BlockSpec((F, F), lambda i: (0, 0)),
            pl.BlockSpec((F, 1), lambda i: (0, 0)),
            pl.BlockSpec((F, 1), lambda i: (0, 0)),
        ],
        out_specs=[
            pl.BlockSpec((blk, F), lambda i: (i, 0)),
            pl.BlockSpec((blk, 1), lambda i: (i, 0)),
            pl.BlockSpec((blk, 1), lambda i: (i, 0)),
        ],
        out_shape=[
            jax.ShapeDtypeStruct((n, F), jnp.float32),
            jax.ShapeDtypeStruct((n, 1), jnp.float32),
            jax.ShapeDtypeStruct((n, 1), jnp.float32),
        ],
    )(o_full, o_full, d0, d1, b1.reshape(1, F), W2, as2.reshape(F, 1),
      ad2.reshape(F, 1))


def _tc_final(o_full, d0, d1, b2, n):
    blk = 2000

    def body(o0_ref, o1_ref, d0_ref, d1_ref, b_ref, out_ref):
        den = d0_ref[...] + d1_ref[...] + jnp.float32(1e-16)
        out_ref[...] = (o0_ref[0] + o1_ref[0]) / den + b_ref[...]

    return pl.pallas_call(
        body,
        grid=(n // blk,),
        in_specs=[
            pl.BlockSpec((1, blk, F), lambda i: (0, i, 0)),
            pl.BlockSpec((1, blk, F), lambda i: (1, i, 0)),
            pl.BlockSpec((blk, 1), lambda i: (i, 0)),
            pl.BlockSpec((blk, 1), lambda i: (i, 0)),
            pl.BlockSpec((1, F), lambda i: (0, 0)),
        ],
        out_specs=pl.BlockSpec((blk, F), lambda i: (i, 0)),
        out_shape=jax.ShapeDtypeStruct((n, F), jnp.float32),
    )(o_full, o_full, d0, d1, b2.reshape(1, F))


def kernel(x, edge_index, batch, emb, W1, as1, ad1, b1, W2, as2, ad2, b2):
    n = x.shape[0]
    e = edge_index.shape[1]
    te = e + n
    unit = NW * CH * NB
    pe = -(-te // unit) * unit
    pad = pe - te
    nr = -(-(n + L) // 256) * 256

    loop = jnp.arange(n, dtype=jnp.int32)
    pad_i = jnp.arange(pad, dtype=jnp.int32)
    src = jnp.concatenate([edge_index[0].astype(jnp.int32), loop, pad_i % n])
    dst = jnp.concatenate(
        [edge_index[1].astype(jnp.int32), loop, n + pad_i % (nr - n)])
    sd = jnp.concatenate(
        [src.reshape(pe // CH, CH), dst.reshape(pe // CH, CH)],
        axis=1).reshape(-1)
    z2 = jnp.zeros((nr // NS, F), jnp.float32)
    z1 = jnp.zeros((nr // NS,), jnp.float32)
    zpad = jnp.zeros((nr - n,), jnp.float32)

    h1, as1n, ad1n = _tc_prep(x.reshape(n, 1).astype(jnp.int32),
                              emb, W1, as1, ad1)
    edge_pass = _sc_edge_pass_real(nr, pe)
    out1, den1 = edge_pass(
        sd, h1, jnp.concatenate([as1n.reshape(n), zpad]),
        jnp.concatenate([ad1n.reshape(n), zpad]), z2, z1)
    hw, asn, adn = _tc_mid(out1, den1[:n, None], den1[nr:nr + n, None],
                           b1, W2, as2, ad2, n)
    out2, den2 = edge_pass(
        sd, hw, jnp.concatenate([asn.reshape(n), zpad]),
        jnp.concatenate([adn.reshape(n), zpad]), z2, z1)
    return _tc_final(out2, den2[:n, None], den2[nr:nr + n, None], b2, n)
